# baseline (speedup 1.0000x reference)
import functools

import jax
import jax.numpy as jnp
from jax import lax
from jax.experimental import pallas as pl
from jax.experimental.pallas import tpu as pltpu
from jax.experimental.pallas import tpu_sc as plsc

BATCH = 16384
EMBED = 128
NC = 2
NS = 16
L = 16
NW = NC * NS
B_PER_W = BATCH // NW
CHUNK = 64
N_CHUNKS = B_PER_W // CHUNK
NSLOT = 4
SEGS = EMBED // L


def _body(u_ids_hbm, i_ids_hbm, usr_hbm, itm_hbm, ent_hbm, pad_hbm, out_hbm,
          *scratch):
    uidx = scratch[0:N_CHUNKS]
    iidx = scratch[N_CHUNKS:2 * N_CHUNKS]
    eidx = scratch[2 * N_CHUNKS:3 * N_CHUNKS]
    urows, irows, erows, score_v = scratch[3 * N_CHUNKS:3 * N_CHUNKS + 4]
    sem_ids = scratch[3 * N_CHUNKS + 4]
    sem_out = scratch[3 * N_CHUNKS + 5]
    slot_sems = scratch[3 * N_CHUNKS + 6:]

    wid = lax.axis_index("s") * NC + lax.axis_index("c")
    base = wid * B_PER_W

    lane = lax.iota(jnp.int32, L)
    sel1 = jnp.bitwise_and(lane, 1) == 0
    sel2 = jnp.bitwise_and(lane, 2) == 0
    lane_div4 = lax.shift_right_logical(lane, 2)

    def shuf(v, k):
        return jnp.take_along_axis(v, jnp.bitwise_xor(lane, k), axis=0)

    id_cps = []
    for c in range(N_CHUNKS):
        off = base + c * CHUNK
        id_cps.append(pltpu.async_copy(u_ids_hbm.at[pl.ds(off, CHUNK)], uidx[c], sem_ids))
        id_cps.append(pltpu.async_copy(i_ids_hbm.at[pl.ds(off, CHUNK)], iidx[c], sem_ids))
    for cp in id_cps:
        cp.wait()
    e_cps = [pltpu.async_copy(pad_hbm.at[iidx[c]], eidx[c], sem_ids)
             for c in range(N_CHUNKS)]

    H = CHUNK // 2

    def fire_ui(c):
        s = c % NSLOT
        sem = slot_sems[s]
        return (
            pltpu.async_copy(usr_hbm.at[uidx[c].at[pl.ds(0, H)]],
                             urows.at[s, pl.ds(0, H)], sem),
            pltpu.async_copy(usr_hbm.at[uidx[c].at[pl.ds(H, H)]],
                             urows.at[s, pl.ds(H, H)], sem),
            pltpu.async_copy(itm_hbm.at[iidx[c].at[pl.ds(0, H)]],
                             irows.at[s, pl.ds(0, H)], sem),
            pltpu.async_copy(itm_hbm.at[iidx[c].at[pl.ds(H, H)]],
                             irows.at[s, pl.ds(H, H)], sem),
        )

    def fire_e(c):
        s = c % NSLOT
        e_cps[c].wait()
        sem = slot_sems[s]
        return (
            pltpu.async_copy(ent_hbm.at[eidx[c].at[pl.ds(0, H)]],
                             erows.at[s, pl.ds(0, H)], sem),
            pltpu.async_copy(ent_hbm.at[eidx[c].at[pl.ds(H, H)]],
                             erows.at[s, pl.ds(H, H)], sem),
        )

    inflight = []
    for c in range(NSLOT - 1):
        cps = fire_ui(c)
        inflight.append(cps + fire_e(c))

    out_cps = []
    for c in range(N_CHUNKS):
        for cp in inflight[0]:
            cp.wait()
        inflight = inflight[1:]
        s = c % NSLOT

        def row_acc(r):
            acc0 = jnp.zeros((L,), jnp.float32)
            acc1 = jnp.zeros((L,), jnp.float32)
            for seg in range(SEGS):
                u = urows[s, r, pl.ds(seg * L, L)]
                iv = irows[s, r, pl.ds(seg * L, L)]
                e = erows[s, r, pl.ds(seg * L, L)]
                if seg % 2 == 0:
                    acc0 = acc0 + u * (iv + e)
                else:
                    acc1 = acc1 + u * (iv + e)
            return acc0 + acc1

        def quad_body(q, score_vec):
            a, b, c4, d = (row_acc(q * 4 + k) for k in range(4))
            ab = jnp.where(sel1, a + shuf(a, 1), b + shuf(b, 1))
            cd = jnp.where(sel1, c4 + shuf(c4, 1), d + shuf(d, 1))
            abcd = jnp.where(sel2, ab + shuf(ab, 2), cd + shuf(cd, 2))
            e4 = abcd + shuf(abcd, 4)
            f = e4 + shuf(e4, 8)
            sub = jnp.bitwise_and(q, 3)
            score_vec = jnp.where(lane_div4 == sub, f, score_vec)

            @pl.when(sub == 3)
            def _store():
                score_v[c, pl.ds((q // 4) * L, L)] = score_vec

            return score_vec

        lax.fori_loop(0, CHUNK // 4, quad_body, jnp.zeros((L,), jnp.float32))

        n = c + NSLOT - 1
        if n < N_CHUNKS:
            cps = fire_ui(n)
            inflight.append(cps + fire_e(n))
        out_cps.append(pltpu.async_copy(
            score_v.at[c], out_hbm.at[pl.ds(base + c * CHUNK, CHUNK)], sem_out))
    for cp in out_cps:
        cp.wait()


@jax.jit
def _run(u_ids, i_ids, usr_emb, itm_emb, ent_emb, padding_items):
    mesh = plsc.VectorSubcoreMesh(core_axis_name="c", subcore_axis_name="s")
    idx_scratch = [pltpu.VMEM((CHUNK,), jnp.int32) for _ in range(3 * N_CHUNKS)]
    return pl.kernel(
        _body,
        mesh=mesh,
        compiler_params=pltpu.CompilerParams(needs_layout_passes=False),
        out_type=jax.ShapeDtypeStruct((BATCH,), jnp.float32),
        scratch_types=idx_scratch + [
            pltpu.VMEM((NSLOT, CHUNK, EMBED), jnp.float32),
            pltpu.VMEM((NSLOT, CHUNK, EMBED), jnp.float32),
            pltpu.VMEM((NSLOT, CHUNK, EMBED), jnp.float32),
            pltpu.VMEM((N_CHUNKS, CHUNK), jnp.float32),
            pltpu.SemaphoreType.DMA,
            pltpu.SemaphoreType.DMA,
        ] + [pltpu.SemaphoreType.DMA for _ in range(NSLOT)],
    )(u_ids, i_ids, usr_emb, itm_emb, ent_emb, padding_items)


def kernel(u_ids, i_ids, usr_emb, itm_emb, ent_emb, padding_items):
    u_ids = jnp.asarray(u_ids, jnp.int32).reshape(BATCH)
    i_ids = jnp.asarray(i_ids, jnp.int32).reshape(BATCH)
    return _run(u_ids, i_ids, usr_emb, itm_emb, ent_emb, padding_items)


# speedup vs baseline: 1.0051x; 1.0051x over previous
import functools

import jax
import jax.numpy as jnp
from jax import lax
from jax.experimental import pallas as pl
from jax.experimental.pallas import tpu as pltpu
from jax.experimental.pallas import tpu_sc as plsc

BATCH = 16384
EMBED = 128
NC = 2
NS = 16
L = 16
NW = NC * NS
B_PER_W = BATCH // NW
CHUNK = 64
N_CHUNKS = B_PER_W // CHUNK
NSLOT = 4
SEGS = EMBED // L


def _body(u_ids_hbm, i_ids_hbm, usr_hbm, itm_hbm, ent_hbm, pad_hbm, out_hbm,
          *scratch):
    uidx = scratch[0:N_CHUNKS]
    iidx = scratch[N_CHUNKS:2 * N_CHUNKS]
    eidx = scratch[2 * N_CHUNKS:3 * N_CHUNKS]
    urows, irows, erows, score_v = scratch[3 * N_CHUNKS:3 * N_CHUNKS + 4]
    sem_ids = scratch[3 * N_CHUNKS + 4]
    sem_out = scratch[3 * N_CHUNKS + 5]
    slot_sems = scratch[3 * N_CHUNKS + 6:]

    wid = lax.axis_index("s") * NC + lax.axis_index("c")
    base = wid * B_PER_W

    lane = lax.iota(jnp.int32, L)
    sel1 = jnp.bitwise_and(lane, 1) == 0
    sel2 = jnp.bitwise_and(lane, 2) == 0
    lane_div4 = lax.shift_right_logical(lane, 2)

    def shuf(v, k):
        return jnp.take_along_axis(v, jnp.bitwise_xor(lane, k), axis=0)

    id_cps = []
    for c in range(N_CHUNKS):
        off = base + c * CHUNK
        id_cps.append(pltpu.async_copy(u_ids_hbm.at[pl.ds(off, CHUNK)], uidx[c], sem_ids))
        id_cps.append(pltpu.async_copy(i_ids_hbm.at[pl.ds(off, CHUNK)], iidx[c], sem_ids))
    for cp in id_cps:
        cp.wait()
    e_cps = [pltpu.async_copy(pad_hbm.at[iidx[c]], eidx[c], sem_ids)
             for c in range(N_CHUNKS)]

    def fire_ui(c):
        s = c % NSLOT
        sem = slot_sems[s]
        return (pltpu.async_copy(usr_hbm.at[uidx[c]], urows.at[s], sem),
                pltpu.async_copy(itm_hbm.at[iidx[c]], irows.at[s], sem))

    def fire_e(c):
        s = c % NSLOT
        e_cps[c].wait()
        return (pltpu.async_copy(ent_hbm.at[eidx[c]], erows.at[s],
                                 slot_sems[s]),)

    inflight = []
    for c in range(NSLOT - 1):
        cps = fire_ui(c)
        inflight.append(cps + fire_e(c))

    out_cps = []
    for c in range(N_CHUNKS):
        for cp in inflight[0]:
            cp.wait()
        inflight = inflight[1:]
        s = c % NSLOT

        def row_acc(r):
            acc0 = jnp.zeros((L,), jnp.float32)
            acc1 = jnp.zeros((L,), jnp.float32)
            for seg in range(SEGS):
                u = urows[s, r, pl.ds(seg * L, L)]
                iv = irows[s, r, pl.ds(seg * L, L)]
                e = erows[s, r, pl.ds(seg * L, L)]
                if seg % 2 == 0:
                    acc0 = acc0 + u * (iv + e)
                else:
                    acc1 = acc1 + u * (iv + e)
            return acc0 + acc1

        def quad_body(q, score_vec):
            a, b, c4, d = (row_acc(q * 4 + k) for k in range(4))
            ab = jnp.where(sel1, a + shuf(a, 1), b + shuf(b, 1))
            cd = jnp.where(sel1, c4 + shuf(c4, 1), d + shuf(d, 1))
            abcd = jnp.where(sel2, ab + shuf(ab, 2), cd + shuf(cd, 2))
            e4 = abcd + shuf(abcd, 4)
            f = e4 + shuf(e4, 8)
            sub = jnp.bitwise_and(q, 3)
            score_vec = jnp.where(lane_div4 == sub, f, score_vec)

            @pl.when(sub == 3)
            def _store():
                score_v[c, pl.ds((q // 4) * L, L)] = score_vec

            return score_vec

        lax.fori_loop(0, CHUNK // 4, quad_body, jnp.zeros((L,), jnp.float32))

        n = c + NSLOT - 1
        if n < N_CHUNKS:
            cps = fire_ui(n)
            inflight.append(cps + fire_e(n))
        out_cps.append(pltpu.async_copy(
            score_v.at[c], out_hbm.at[pl.ds(base + c * CHUNK, CHUNK)], sem_out))
    for cp in out_cps:
        cp.wait()


@jax.jit
def _run(u_ids, i_ids, usr_emb, itm_emb, ent_emb, padding_items):
    mesh = plsc.VectorSubcoreMesh(core_axis_name="c", subcore_axis_name="s")
    idx_scratch = [pltpu.VMEM((CHUNK,), jnp.int32) for _ in range(3 * N_CHUNKS)]
    return pl.kernel(
        _body,
        mesh=mesh,
        compiler_params=pltpu.CompilerParams(needs_layout_passes=False,
                                             skip_device_barrier=True),
        out_type=jax.ShapeDtypeStruct((BATCH,), jnp.float32),
        scratch_types=idx_scratch + [
            pltpu.VMEM((NSLOT, CHUNK, EMBED), jnp.float32),
            pltpu.VMEM((NSLOT, CHUNK, EMBED), jnp.float32),
            pltpu.VMEM((NSLOT, CHUNK, EMBED), jnp.float32),
            pltpu.VMEM((N_CHUNKS, CHUNK), jnp.float32),
            pltpu.SemaphoreType.DMA,
            pltpu.SemaphoreType.DMA,
        ] + [pltpu.SemaphoreType.DMA for _ in range(NSLOT)],
    )(u_ids, i_ids, usr_emb, itm_emb, ent_emb, padding_items)


def kernel(u_ids, i_ids, usr_emb, itm_emb, ent_emb, padding_items):
    u_ids = jnp.asarray(u_ids, jnp.int32).reshape(BATCH)
    i_ids = jnp.asarray(i_ids, jnp.int32).reshape(BATCH)
    return _run(u_ids, i_ids, usr_emb, itm_emb, ent_emb, padding_items)


# speedup vs baseline: 1.0489x; 1.0436x over previous
import functools

import jax
import jax.numpy as jnp
from jax import lax
from jax.experimental import pallas as pl
from jax.experimental.pallas import tpu as pltpu
from jax.experimental.pallas import tpu_sc as plsc

BATCH = 16384
EMBED = 128
NC = 2
NS = 16
L = 16
NW = NC * NS
B_PER_W = BATCH // NW
CHUNK = 64
N_CHUNKS = B_PER_W // CHUNK
NSLOT = 4
SEGS = EMBED // L


def _body(u_ids_hbm, i_ids_hbm, usr_hbm, itm_hbm, ent_hbm, pad_hbm, out_hbm,
          *scratch):
    uidx = scratch[0:N_CHUNKS]
    iidx = scratch[N_CHUNKS:2 * N_CHUNKS]
    eidx = scratch[2 * N_CHUNKS:3 * N_CHUNKS]
    urows, irows, score_v = scratch[3 * N_CHUNKS:3 * N_CHUNKS + 3]
    sem_ids = scratch[3 * N_CHUNKS + 3]
    sem_out = scratch[3 * N_CHUNKS + 4]
    rest = scratch[3 * N_CHUNKS + 5:]
    u_sems = rest[:NSLOT]
    i_sems = rest[NSLOT:]

    wid = lax.axis_index("s") * NC + lax.axis_index("c")
    base = wid * B_PER_W

    lane = lax.iota(jnp.int32, L)
    sel1 = jnp.bitwise_and(lane, 1) == 0
    sel2 = jnp.bitwise_and(lane, 2) == 0
    lane_div4 = lax.shift_right_logical(lane, 2)

    def shuf(v, k):
        return jnp.take_along_axis(v, jnp.bitwise_xor(lane, k), axis=0)

    id_cps = []
    for c in range(N_CHUNKS):
        off = base + c * CHUNK
        id_cps.append(pltpu.async_copy(u_ids_hbm.at[pl.ds(off, CHUNK)], uidx[c], sem_ids))
        id_cps.append(pltpu.async_copy(i_ids_hbm.at[pl.ds(off, CHUNK)], iidx[c], sem_ids))
    for cp in id_cps:
        cp.wait()
    e_cps = [pltpu.async_copy(pad_hbm.at[iidx[c]], eidx[c], sem_ids)
             for c in range(N_CHUNKS)]

    def fire_ui(c):
        s = c % NSLOT
        return (pltpu.async_copy(usr_hbm.at[uidx[c]], urows.at[s], u_sems[s]),
                pltpu.async_copy(itm_hbm.at[iidx[c]], irows.at[s], i_sems[s]))

    def fire_eadd(c, i_cp):
        i_cp.wait()
        s = c % NSLOT
        e_cps[c].wait()
        return pltpu.async_copy(ent_hbm.at[eidx[c]], irows.at[s], i_sems[s],
                                add=True)

    ui = [fire_ui(c) for c in range(min(NSLOT - 1, N_CHUNKS))]
    ie = [fire_eadd(c, ui[c][1]) for c in range(min(2, len(ui)))]

    out_cps = []
    for c in range(N_CHUNKS):
        ui[c][0].wait()
        ie[c].wait()
        s = c % NSLOT

        def row_acc(r):
            acc0 = jnp.zeros((L,), jnp.float32)
            acc1 = jnp.zeros((L,), jnp.float32)
            for seg in range(SEGS):
                u = urows[s, r, pl.ds(seg * L, L)]
                ie_v = irows[s, r, pl.ds(seg * L, L)]
                if seg % 2 == 0:
                    acc0 = acc0 + u * ie_v
                else:
                    acc1 = acc1 + u * ie_v
            return acc0 + acc1

        def quad_body(q, score_vec):
            a, b, c4, d = (row_acc(q * 4 + k) for k in range(4))
            ab = jnp.where(sel1, a + shuf(a, 1), b + shuf(b, 1))
            cd = jnp.where(sel1, c4 + shuf(c4, 1), d + shuf(d, 1))
            abcd = jnp.where(sel2, ab + shuf(ab, 2), cd + shuf(cd, 2))
            e4 = abcd + shuf(abcd, 4)
            f = e4 + shuf(e4, 8)
            sub = jnp.bitwise_and(q, 3)
            score_vec = jnp.where(lane_div4 == sub, f, score_vec)

            @pl.when(sub == 3)
            def _store():
                score_v[c, pl.ds((q // 4) * L, L)] = score_vec

            return score_vec

        lax.fori_loop(0, CHUNK // 4, quad_body, jnp.zeros((L,), jnp.float32))

        n = c + NSLOT - 1
        if n < N_CHUNKS:
            ui.append(fire_ui(n))
        m = c + 2
        if m < N_CHUNKS:
            ie.append(fire_eadd(m, ui[m][1]))
        out_cps.append(pltpu.async_copy(
            score_v.at[c], out_hbm.at[pl.ds(base + c * CHUNK, CHUNK)], sem_out))
    for cp in out_cps:
        cp.wait()


@jax.jit
def _run(u_ids, i_ids, usr_emb, itm_emb, ent_emb, padding_items):
    mesh = plsc.VectorSubcoreMesh(core_axis_name="c", subcore_axis_name="s")
    idx_scratch = [pltpu.VMEM((CHUNK,), jnp.int32) for _ in range(3 * N_CHUNKS)]
    return pl.kernel(
        _body,
        mesh=mesh,
        compiler_params=pltpu.CompilerParams(needs_layout_passes=False),
        out_type=jax.ShapeDtypeStruct((BATCH,), jnp.float32),
        scratch_types=idx_scratch + [
            pltpu.VMEM((NSLOT, CHUNK, EMBED), jnp.float32),
            pltpu.VMEM((NSLOT, CHUNK, EMBED), jnp.float32),
            pltpu.VMEM((N_CHUNKS, CHUNK), jnp.float32),
            pltpu.SemaphoreType.DMA,
            pltpu.SemaphoreType.DMA,
        ] + [pltpu.SemaphoreType.DMA for _ in range(2 * NSLOT)],
    )(u_ids, i_ids, usr_emb, itm_emb, ent_emb, padding_items)


def kernel(u_ids, i_ids, usr_emb, itm_emb, ent_emb, padding_items):
    u_ids = jnp.asarray(u_ids, jnp.int32).reshape(BATCH)
    i_ids = jnp.asarray(i_ids, jnp.int32).reshape(BATCH)
    return _run(u_ids, i_ids, usr_emb, itm_emb, ent_emb, padding_items)


# speedup vs baseline: 1.0819x; 1.0315x over previous
import functools

import jax
import jax.numpy as jnp
from jax import lax
from jax.experimental import pallas as pl
from jax.experimental.pallas import tpu as pltpu
from jax.experimental.pallas import tpu_sc as plsc

BATCH = 16384
EMBED = 128
NC = 2
NS = 16
L = 16
NW = NC * NS
B_PER_W = BATCH // NW
CHUNK = 64
N_CHUNKS = B_PER_W // CHUNK
NSLOT = 6
EDEPTH = 3
SEGS = EMBED // L


def _body(u_ids_hbm, i_ids_hbm, usr_hbm, itm_hbm, ent_hbm, pad_hbm, out_hbm,
          *scratch):
    uidx = scratch[0:N_CHUNKS]
    iidx = scratch[N_CHUNKS:2 * N_CHUNKS]
    eidx = scratch[2 * N_CHUNKS:3 * N_CHUNKS]
    urows, irows, score_v = scratch[3 * N_CHUNKS:3 * N_CHUNKS + 3]
    sem_ids = scratch[3 * N_CHUNKS + 3]
    sem_out = scratch[3 * N_CHUNKS + 4]
    rest = scratch[3 * N_CHUNKS + 5:]
    u_sems = rest[:NSLOT]
    i_sems = rest[NSLOT:]

    wid = lax.axis_index("s") * NC + lax.axis_index("c")
    base = wid * B_PER_W

    lane = lax.iota(jnp.int32, L)
    sel1 = jnp.bitwise_and(lane, 1) == 0
    sel2 = jnp.bitwise_and(lane, 2) == 0
    lane_div4 = lax.shift_right_logical(lane, 2)

    def shuf(v, k):
        return jnp.take_along_axis(v, jnp.bitwise_xor(lane, k), axis=0)

    id_cps = []
    for c in range(N_CHUNKS):
        off = base + c * CHUNK
        id_cps.append(pltpu.async_copy(u_ids_hbm.at[pl.ds(off, CHUNK)], uidx[c], sem_ids))
        id_cps.append(pltpu.async_copy(i_ids_hbm.at[pl.ds(off, CHUNK)], iidx[c], sem_ids))
    for cp in id_cps:
        cp.wait()
    e_cps = [pltpu.async_copy(pad_hbm.at[iidx[c]], eidx[c], sem_ids)
             for c in range(N_CHUNKS)]

    def fire_ui(c):
        s = c % NSLOT
        return (pltpu.async_copy(usr_hbm.at[uidx[c]], urows.at[s], u_sems[s]),
                pltpu.async_copy(itm_hbm.at[iidx[c]], irows.at[s], i_sems[s]))

    def fire_eadd(c, i_cp):
        i_cp.wait()
        s = c % NSLOT
        e_cps[c].wait()
        return pltpu.async_copy(ent_hbm.at[eidx[c]], irows.at[s], i_sems[s],
                                add=True)

    ui = [fire_ui(c) for c in range(min(NSLOT - 1, N_CHUNKS))]
    ie = [fire_eadd(c, ui[c][1]) for c in range(min(EDEPTH, len(ui)))]

    out_cps = []
    for c in range(N_CHUNKS):
        ui[c][0].wait()
        ie[c].wait()
        s = c % NSLOT

        def row_acc(r):
            acc0 = jnp.zeros((L,), jnp.float32)
            acc1 = jnp.zeros((L,), jnp.float32)
            for seg in range(SEGS):
                u = urows[s, r, pl.ds(seg * L, L)]
                ie_v = irows[s, r, pl.ds(seg * L, L)]
                if seg % 2 == 0:
                    acc0 = acc0 + u * ie_v
                else:
                    acc1 = acc1 + u * ie_v
            return acc0 + acc1

        def quad_body(q, score_vec):
            a, b, c4, d = (row_acc(q * 4 + k) for k in range(4))
            ab = jnp.where(sel1, a + shuf(a, 1), b + shuf(b, 1))
            cd = jnp.where(sel1, c4 + shuf(c4, 1), d + shuf(d, 1))
            abcd = jnp.where(sel2, ab + shuf(ab, 2), cd + shuf(cd, 2))
            e4 = abcd + shuf(abcd, 4)
            f = e4 + shuf(e4, 8)
            sub = jnp.bitwise_and(q, 3)
            score_vec = jnp.where(lane_div4 == sub, f, score_vec)

            @pl.when(sub == 3)
            def _store():
                score_v[c, pl.ds((q // 4) * L, L)] = score_vec

            return score_vec

        lax.fori_loop(0, CHUNK // 4, quad_body, jnp.zeros((L,), jnp.float32))

        n = c + NSLOT - 1
        if n < N_CHUNKS:
            ui.append(fire_ui(n))
        m = c + EDEPTH
        if m < N_CHUNKS:
            ie.append(fire_eadd(m, ui[m][1]))
        out_cps.append(pltpu.async_copy(
            score_v.at[c], out_hbm.at[pl.ds(base + c * CHUNK, CHUNK)], sem_out))
    for cp in out_cps:
        cp.wait()


@jax.jit
def _run(u_ids, i_ids, usr_emb, itm_emb, ent_emb, padding_items):
    mesh = plsc.VectorSubcoreMesh(core_axis_name="c", subcore_axis_name="s")
    idx_scratch = [pltpu.VMEM((CHUNK,), jnp.int32) for _ in range(3 * N_CHUNKS)]
    return pl.kernel(
        _body,
        mesh=mesh,
        compiler_params=pltpu.CompilerParams(needs_layout_passes=False),
        out_type=jax.ShapeDtypeStruct((BATCH,), jnp.float32),
        scratch_types=idx_scratch + [
            pltpu.VMEM((NSLOT, CHUNK, EMBED), jnp.float32),
            pltpu.VMEM((NSLOT, CHUNK, EMBED), jnp.float32),
            pltpu.VMEM((N_CHUNKS, CHUNK), jnp.float32),
            pltpu.SemaphoreType.DMA,
            pltpu.SemaphoreType.DMA,
        ] + [pltpu.SemaphoreType.DMA for _ in range(2 * NSLOT)],
    )(u_ids, i_ids, usr_emb, itm_emb, ent_emb, padding_items)


def kernel(u_ids, i_ids, usr_emb, itm_emb, ent_emb, padding_items):
    u_ids = jnp.asarray(u_ids, jnp.int32).reshape(BATCH)
    i_ids = jnp.asarray(i_ids, jnp.int32).reshape(BATCH)
    return _run(u_ids, i_ids, usr_emb, itm_emb, ent_emb, padding_items)
